# baseline (speedup 1.0000x reference)
import functools

import jax
import jax.numpy as jnp
from jax import lax
from jax.experimental import pallas as pl
from jax.experimental.pallas import tpu as pltpu
from jax.experimental.pallas import tpu_sc as plsc

RANK = 32
NMODE = 3
LANES = 16
VROW = 128
RPV = VROW // RANK
CHUNK = 128

_info = plsc.get_sparse_core_info()
_NC, _NS = _info.num_cores, _info.num_subcores
_NW = _NC * _NS


def _make_kernel(batch: int, nrows: int):
    bpw = batch // _NW
    nchunk = bpw // CHUNK
    groups = CHUNK // LANES

    mesh = plsc.VectorSubcoreMesh(core_axis_name="c", subcore_axis_name="s")

    @functools.partial(
        pl.kernel,
        mesh=mesh,
        out_type=jax.ShapeDtypeStruct((batch,), jnp.float32),
        compiler_params=pltpu.CompilerParams(
            needs_layout_passes=False, use_tc_tiling_on_sc=True),
        scratch_types=[
            pltpu.VMEM((bpw * NMODE,), jnp.int32),
            pltpu.VMEM((bpw,), jnp.int32),
            pltpu.VMEM((bpw,), jnp.int32),
            pltpu.VMEM((bpw,), jnp.int32),
            pltpu.VMEM((bpw,), jnp.int32),
            pltpu.VMEM((bpw,), jnp.int32),
            pltpu.VMEM((bpw,), jnp.int32),
            pltpu.VMEM((2, CHUNK, VROW), jnp.float32),
            pltpu.VMEM((2, CHUNK, VROW), jnp.float32),
            pltpu.VMEM((2, CHUNK, VROW), jnp.float32),
            pltpu.VMEM((bpw,), jnp.float32),
            pltpu.SemaphoreType.DMA,
            pltpu.SemaphoreType.DMA,
            pltpu.SemaphoreType.DMA,
        ],
    )
    def cpd_kernel(idx_hbm, e0_hbm, e1_hbm, e2_hbm, out_hbm,
                   iflat_v, c0_v, c1_v, c2_v, k0_v, k1_v, k2_v,
                   r0_v, r1_v, r2_v, out_v,
                   sem0, sem1, sem2):
        wid = lax.axis_index("s") * _NC + lax.axis_index("c")
        base = wid * bpw

        pltpu.sync_copy(idx_hbm.at[pl.ds(base * NMODE, bpw * NMODE)], iflat_v)

        lane = lax.iota(jnp.int32, LANES)

        def deint_body(g, carry):
            flat0 = (g * LANES + lane) * NMODE
            sl = pl.ds(g * LANES, LANES)
            for m, (cm_v, km_v) in enumerate(
                    ((c0_v, k0_v), (c1_v, k1_v), (c2_v, k2_v))):
                i = plsc.load_gather(iflat_v, [flat0 + m])
                cm_v[sl] = (i & (RPV - 1)) * RANK
                km_v[sl] = i >> 2
            return carry

        lax.fori_loop(0, bpw // LANES, deint_body, 0)

        tabs = ((e0_hbm, k0_v, c0_v, r0_v, sem0),
                (e1_hbm, k1_v, c1_v, r1_v, sem1),
                (e2_hbm, k2_v, c2_v, r2_v, sem2))

        def fire(chunk):
            buf = chunk % 2
            cps = []
            for e_hbm, km_v, _, rm_v, sem in tabs:
                cps.append(pltpu.async_copy(
                    e_hbm.at[km_v.at[pl.ds(chunk * CHUNK, CHUNK)]],
                    rm_v.at[buf], sem))
            return cps

        inflight = fire(0)

        for chunk in range(nchunk):
            nxt = fire(chunk + 1) if chunk + 1 < nchunk else []
            for cp in inflight:
                cp.wait()
            inflight = nxt
            buf = chunk % 2

            def red_body(g, carry, buf=buf, chunk=chunk):
                b0 = chunk * CHUNK + g * LANES
                rows = g * LANES + lane
                sl = pl.ds(b0, LANES)
                cb0 = c0_v[sl]
                cb1 = c1_v[sl]
                cb2 = c2_v[sl]
                acc = jnp.zeros((LANES,), jnp.float32)
                for r in range(RANK):
                    acc = acc + (
                        plsc.load_gather(r0_v.at[buf], [rows, cb0 + r])
                        * plsc.load_gather(r1_v.at[buf], [rows, cb1 + r])
                        * plsc.load_gather(r2_v.at[buf], [rows, cb2 + r]))
                out_v[sl] = acc
                return carry

            lax.fori_loop(0, groups, red_body, 0)

        pltpu.sync_copy(out_v, out_hbm.at[pl.ds(base, bpw)])

    return cpd_kernel


def kernel(idxs, E0, E1, E2):
    batch = idxs.shape[0]
    nrows = E0.shape[0]
    vrows = nrows * RANK // VROW
    idx_flat = idxs.astype(jnp.int32).reshape(batch * NMODE)
    e0 = E0.reshape(vrows, VROW)
    e1 = E1.reshape(vrows, VROW)
    e2 = E2.reshape(vrows, VROW)
    return _make_kernel(batch, nrows)(idx_flat, e0, e1, e2)


# speedup vs baseline: 1.0246x; 1.0246x over previous
import functools

import jax
import jax.numpy as jnp
from jax import lax
from jax.experimental import pallas as pl
from jax.experimental.pallas import tpu as pltpu
from jax.experimental.pallas import tpu_sc as plsc

RANK = 32
NMODE = 3
LANES = 16

_info = plsc.get_sparse_core_info()
_NC, _NS = _info.num_cores, _info.num_subcores
_NW = _NC * _NS


def _make_kernel(batch: int):
    bpw = batch // _NW

    mesh = plsc.VectorSubcoreMesh(core_axis_name="c", subcore_axis_name="s")

    @functools.partial(
        pl.kernel,
        mesh=mesh,
        out_type=jax.ShapeDtypeStruct((batch,), jnp.float32),
        compiler_params=pltpu.CompilerParams(
            needs_layout_passes=False, use_tc_tiling_on_sc=False),
        scratch_types=[
            pltpu.VMEM((bpw * NMODE,), jnp.int32),
            pltpu.VMEM((bpw,), jnp.int32),
            pltpu.VMEM((bpw,), jnp.int32),
            pltpu.VMEM((bpw,), jnp.int32),
            pltpu.VMEM((bpw, RANK), jnp.float32),
            pltpu.VMEM((bpw, RANK), jnp.float32),
            pltpu.VMEM((bpw, RANK), jnp.float32),
            pltpu.VMEM((bpw,), jnp.float32),
            pltpu.SemaphoreType.DMA,
            pltpu.SemaphoreType.DMA,
            pltpu.SemaphoreType.DMA,
        ],
    )
    def cpd_kernel(idx_hbm, e0_hbm, e1_hbm, e2_hbm, out_hbm,
                   iflat_v, i0_v, i1_v, i2_v, r0_v, r1_v, r2_v, out_v,
                   sem0, sem1, sem2):
        wid = lax.axis_index("s") * _NC + lax.axis_index("c")
        base = wid * bpw

        pltpu.sync_copy(idx_hbm.at[pl.ds(base * NMODE, bpw * NMODE)], iflat_v)

        lane = lax.iota(jnp.int32, LANES)

        def deint_body(g, carry):
            flat0 = (g * LANES + lane) * NMODE
            i0_v[pl.ds(g * LANES, LANES)] = plsc.load_gather(iflat_v, [flat0])
            i1_v[pl.ds(g * LANES, LANES)] = plsc.load_gather(iflat_v, [flat0 + 1])
            i2_v[pl.ds(g * LANES, LANES)] = plsc.load_gather(iflat_v, [flat0 + 2])
            return carry

        lax.fori_loop(0, bpw // LANES, deint_body, 0)

        cp0 = pltpu.async_copy(e0_hbm.at[i0_v], r0_v, sem0)
        cp1 = pltpu.async_copy(e1_hbm.at[i1_v], r1_v, sem1)
        cp2 = pltpu.async_copy(e2_hbm.at[i2_v], r2_v, sem2)
        cp0.wait()
        cp1.wait()
        cp2.wait()

        def group_body(g, carry):
            b0 = g * LANES
            acc = jnp.zeros((LANES,), jnp.float32)
            for j in range(LANES):
                b = b0 + j
                lo = (r0_v[b, pl.ds(0, LANES)] * r1_v[b, pl.ds(0, LANES)]
                      * r2_v[b, pl.ds(0, LANES)])
                hi = (r0_v[b, pl.ds(LANES, LANES)] * r1_v[b, pl.ds(LANES, LANES)]
                      * r2_v[b, pl.ds(LANES, LANES)])
                acc = jnp.where(lane == j, jnp.sum(lo + hi), acc)
            out_v[pl.ds(b0, LANES)] = acc
            return carry

        lax.fori_loop(0, bpw // LANES, group_body, 0)

        pltpu.sync_copy(out_v, out_hbm.at[pl.ds(base, bpw)])

    return cpd_kernel


def kernel(idxs, E0, E1, E2):
    batch = idxs.shape[0]
    idx_flat = idxs.astype(jnp.int32).reshape(batch * NMODE)
    return _make_kernel(batch)(idx_flat, E0, E1, E2)
